# baseline (speedup 1.0000x reference)
import jax
import jax.numpy as jnp
from jax.experimental import pallas as pl
from jax.experimental.pallas import tpu as pltpu

H = 128
NIN = 48
NMID = 6 * H
TN = 8192


def _critic_body(x_ref, wac_ref, w3_ref, b_ref, o_ref):
    xf = x_ref[...]
    xb = xf.astype(jnp.bfloat16)
    ycore = (jnp.dot(wac_ref[...], xb, preferred_element_type=jnp.float32)
             + b_ref[3 * H:NMID, 0:1])
    ycb = jnp.maximum(ycore.astype(jnp.bfloat16), jnp.bfloat16(0.0))
    outs = []
    for row, wc, bc in ((24, 4, 0), (25, 5, 1), (26, 6, 2)):
        xr = xf[row:row + 1, :]
        s = b_ref[0:H, wc:wc + 1] * xr + b_ref[bc * H:(bc + 1) * H, 0:1]
        outs.append(jnp.maximum(s, 0.0).astype(jnp.bfloat16))
    yb = jnp.concatenate(outs + [ycb], axis=0)
    h = (jnp.dot(w3_ref[...], yb, preferred_element_type=jnp.float32)
         + b_ref[0:H, 1:2])
    hb = jnp.maximum(h, 0.0).astype(jnp.bfloat16)
    wrow = b_ref[0:H, 2:3].astype(jnp.bfloat16).reshape(1, H)
    o = jnp.dot(wrow, hb, preferred_element_type=jnp.float32)
    o_ref[...] = o + b_ref[0:1, 3:4]


def _compose_branch_t(a_w, b_w):
    A = a_w[:, 0, :]
    C = jnp.einsum("dct,ck->dtk", b_w, A)
    return sum(jnp.pad(C[:, t, :], ((0, 0), (t, 4 - t)))
               for t in range(5))


def _prep(fc1_w, fc1_b, fc2_w, fc2_b, fc3_w, fc3_b, c1a_w, c1a_b, c1b_w,
          c1b_b, c2a_w, c2a_b, c2b_w, c2b_b, c3_w, c3_b, o1_w, o1_b,
          o2_w, o2_b):
    M1 = _compose_branch_t(c1a_w, c1b_w)
    M2 = _compose_branch_t(c2a_w, c2b_w)
    bc1 = jnp.sum(c1b_w, axis=2) @ c1a_b + c1b_b
    bc2 = jnp.sum(c2b_w, axis=2) @ c2a_b + c2b_b

    wac = jnp.concatenate([
        jnp.pad(c3_w[:, 0, :], ((0, 0), (16, 5))),
        jnp.pad(M1, ((0, 0), (0, 19))),
        jnp.pad(M2, ((0, 0), (8, 11))),
    ], axis=0)

    w3 = jnp.concatenate([o1_w[:, j * H:(j + 1) * H]
                          for j in (0, 1, 5, 4, 2, 3)], axis=1)

    z = jnp.zeros((H,), jnp.float32)
    top = jnp.stack([fc1_b, o1_b, o2_w[0, :],
                     jnp.pad(o2_b, (0, H - 1)),
                     fc1_w[:, 0], fc2_w[:, 0], fc3_w[:, 0], z], axis=1)
    rest = jnp.stack([jnp.concatenate([fc2_b, fc3_b, c3_b, bc1, bc2]),
                      *([jnp.zeros((5 * H,), jnp.float32)] * 7)], axis=1)
    bias = jnp.concatenate([top, rest], axis=0)

    return wac.astype(jnp.bfloat16), w3.astype(jnp.bfloat16), bias


@jax.jit
def kernel(x, fc1_w, fc1_b, fc2_w, fc2_b, fc3_w, fc3_b, c1a_w, c1a_b, c1b_w,
           c1b_b, c2a_w, c2a_b, c2b_w, c2b_b, c3_w, c3_b, o1_w, o1_b,
           o2_w, o2_b):
    B = x.shape[0]
    wac, w3, bias = _prep(fc1_w, fc1_b, fc2_w, fc2_b, fc3_w, fc3_b, c1a_w,
                          c1a_b, c1b_w, c1b_b, c2a_w, c2a_b, c2b_w, c2b_b,
                          c3_w, c3_b, o1_w, o1_b, o2_w, o2_b)

    xT = jnp.concatenate([x[:, 2, :], x[:, 3, :], x[:, 4, :],
                          x[:, 0, 7:8], x[:, 1, 7:8], x[:, 5, 7:8]],
                         axis=1).T
    tn = TN if B % TN == 0 else 128
    b_pad = ((B + tn - 1) // tn) * tn
    if b_pad != B:
        xT = jnp.pad(xT, ((0, 0), (0, b_pad - B)))

    out = pl.pallas_call(
        _critic_body,
        out_shape=jax.ShapeDtypeStruct((1, b_pad), jnp.float32),
        grid=(b_pad // tn,),
        in_specs=[
            pl.BlockSpec((27, tn), lambda i: (0, i)),
            pl.BlockSpec(wac.shape, lambda i: (0, 0)),
            pl.BlockSpec(w3.shape, lambda i: (0, 0)),
            pl.BlockSpec(bias.shape, lambda i: (0, 0)),
        ],
        out_specs=pl.BlockSpec((1, tn), lambda i: (0, i)),
        compiler_params=pltpu.CompilerParams(
            dimension_semantics=("arbitrary",)),
    )(xT, wac, w3, bias)
    return out[:, :B].reshape(B, 1)


# speedup vs baseline: 1.5486x; 1.5486x over previous
import jax
import jax.numpy as jnp
from jax.experimental import pallas as pl
from jax.experimental.pallas import tpu as pltpu

H = 128
NIN = 48
NMID = 6 * H
TN = 8192


def _critic_body(x_ref, wac_ref, w3_ref, b_ref, o_ref):
    xf = x_ref[...]
    xb = xf.astype(jnp.bfloat16)
    ycore = (jnp.dot(wac_ref[...], xb, preferred_element_type=jnp.float32)
             + b_ref[3 * H:NMID, 0:1])
    ycb = jnp.maximum(ycore.astype(jnp.bfloat16), jnp.bfloat16(0.0))
    outs = []
    for row, wc, bc in ((7, 4, 0), (15, 5, 1), (47, 6, 2)):
        xr = xf[row:row + 1, :]
        s = b_ref[0:H, wc:wc + 1] * xr + b_ref[bc * H:(bc + 1) * H, 0:1]
        outs.append(jnp.maximum(s, 0.0).astype(jnp.bfloat16))
    yb = jnp.concatenate(outs + [ycb], axis=0)
    h = (jnp.dot(w3_ref[...], yb, preferred_element_type=jnp.float32)
         + b_ref[0:H, 1:2])
    hb = jnp.maximum(h, 0.0).astype(jnp.bfloat16)
    wrow = b_ref[0:H, 2:3].astype(jnp.bfloat16).reshape(1, H)
    o = jnp.dot(wrow, hb, preferred_element_type=jnp.float32)
    o_ref[...] = o + b_ref[0:1, 3:4]


def _compose_branch_t(a_w, b_w):
    A = a_w[:, 0, :]
    C = jnp.einsum("dct,ck->dtk", b_w, A)
    return sum(jnp.pad(C[:, t, :], ((0, 0), (t, 4 - t)))
               for t in range(5))


def _prep(fc1_w, fc1_b, fc2_w, fc2_b, fc3_w, fc3_b, c1a_w, c1a_b, c1b_w,
          c1b_b, c2a_w, c2a_b, c2b_w, c2b_b, c3_w, c3_b, o1_w, o1_b,
          o2_w, o2_b):
    M1 = _compose_branch_t(c1a_w, c1b_w)
    M2 = _compose_branch_t(c2a_w, c2b_w)
    bc1 = jnp.sum(c1b_w, axis=2) @ c1a_b + c1b_b
    bc2 = jnp.sum(c2b_w, axis=2) @ c2a_b + c2b_b

    wac = jnp.concatenate([
        jnp.pad(c3_w[:, 0, :], ((0, 0), (32, 10))),
        jnp.pad(M1, ((0, 0), (16, 24))),
        jnp.pad(M2, ((0, 0), (24, 16))),
    ], axis=0)

    w3 = jnp.concatenate([o1_w[:, j * H:(j + 1) * H]
                          for j in (0, 1, 5, 4, 2, 3)], axis=1)

    z = jnp.zeros((H,), jnp.float32)
    top = jnp.stack([fc1_b, o1_b, o2_w[0, :],
                     jnp.pad(o2_b, (0, H - 1)),
                     fc1_w[:, 0], fc2_w[:, 0], fc3_w[:, 0], z], axis=1)
    rest = jnp.stack([jnp.concatenate([fc2_b, fc3_b, c3_b, bc1, bc2]),
                      *([jnp.zeros((5 * H,), jnp.float32)] * 7)], axis=1)
    bias = jnp.concatenate([top, rest], axis=0)

    return wac.astype(jnp.bfloat16), w3.astype(jnp.bfloat16), bias


@jax.jit
def kernel(x, fc1_w, fc1_b, fc2_w, fc2_b, fc3_w, fc3_b, c1a_w, c1a_b, c1b_w,
           c1b_b, c2a_w, c2a_b, c2b_w, c2b_b, c3_w, c3_b, o1_w, o1_b,
           o2_w, o2_b):
    B = x.shape[0]
    wac, w3, bias = _prep(fc1_w, fc1_b, fc2_w, fc2_b, fc3_w, fc3_b, c1a_w,
                          c1a_b, c1b_w, c1b_b, c2a_w, c2a_b, c2b_w, c2b_b,
                          c3_w, c3_b, o1_w, o1_b, o2_w, o2_b)

    xT = x.reshape(B, NIN).T
    tn = TN if B % TN == 0 else 128
    b_pad = ((B + tn - 1) // tn) * tn
    if b_pad != B:
        xT = jnp.pad(xT, ((0, 0), (0, b_pad - B)))

    out = pl.pallas_call(
        _critic_body,
        out_shape=jax.ShapeDtypeStruct((1, b_pad), jnp.float32),
        grid=(b_pad // tn,),
        in_specs=[
            pl.BlockSpec((NIN, tn), lambda i: (0, i)),
            pl.BlockSpec(wac.shape, lambda i: (0, 0)),
            pl.BlockSpec(w3.shape, lambda i: (0, 0)),
            pl.BlockSpec(bias.shape, lambda i: (0, 0)),
        ],
        out_specs=pl.BlockSpec((1, tn), lambda i: (0, i)),
        compiler_params=pltpu.CompilerParams(
            dimension_semantics=("arbitrary",)),
    )(xT, wac, w3, bias)
    return out[:, :B].reshape(B, 1)
